# baseline (speedup 1.0000x reference)
import functools

import jax
import jax.numpy as jnp
from jax import lax
from jax.experimental import pallas as pl
from jax.experimental.pallas import tpu as pltpu, tpu_sc as plsc

B = 1024
V = 100000
NC = 2
NS = 16
L = 16
NW = NC * NS
RPW = B // NW

_mesh = plsc.VectorSubcoreMesh(
    core_axis_name="c", subcore_axis_name="s", num_cores=NC, num_subcores=NS
)

_DN = lax.GatherDimensionNumbers(
    offset_dims=(), collapsed_slice_dims=(0,), start_index_map=(0,)
)


def _shuf(v, idx):
    return lax.gather(
        v, idx.reshape(L, 1), _DN, slice_sizes=(1,),
        mode=lax.GatherScatterMode.PROMISE_IN_BOUNDS,
    )


@functools.partial(
    pl.kernel,
    out_type=jax.ShapeDtypeStruct((B,), jnp.float32),
    mesh=_mesh,
    scratch_types=[
        pltpu.VMEM((V + L,), jnp.float32),
        pltpu.VMEM((RPW + L,), jnp.int32),
        pltpu.VMEM((RPW + L,), jnp.float32),
    ],
)
def _ll_kernel(in_hbm, tg_hbm, out_hbm, row_v, t_v, o_v):
    wid = lax.axis_index("s") * NC + lax.axis_index("c")
    base_row = wid * RPW
    lanes = lax.broadcasted_iota(jnp.int32, (L,), 0)
    zeros_i = lanes * 0
    neg_inf = jnp.full((L,), -jnp.inf, jnp.float32)

    pltpu.sync_copy(tg_hbm.at[pl.ds(base_row, RPW)], t_v.at[pl.ds(0, RPW)])

    def do_row(r, carry):
        rowbase = (base_row + r) * V
        pltpu.sync_copy(in_hbm.at[pl.ds(rowbase, V)], row_v.at[pl.ds(0, V)])
        t = t_v[pl.ds(r, L)][0]
        vec = row_v[pl.ds(t, L)]
        row_v[pl.ds(t, L)] = jnp.where(lanes == 0, neg_inf, vec)
        acc = lax.fori_loop(
            0,
            V // L,
            lambda i, a: jnp.maximum(a, row_v[pl.ds(i * L, L)]),
            neg_inf,
            unroll=8,
        )
        for k in (8, 4, 2, 1):
            acc = jnp.maximum(acc, _shuf(acc, lanes ^ k))
        res = _shuf(vec, zeros_i) - acc
        o_v[pl.ds(r, L)] = res
        return carry

    lax.fori_loop(0, RPW, do_row, 0)
    pltpu.sync_copy(o_v.at[pl.ds(0, RPW)], out_hbm.at[pl.ds(base_row, RPW)])


def kernel(inputs, targets):
    return _ll_kernel(inputs.reshape(-1), targets.astype(jnp.int32))


# speedup vs baseline: 1.1137x; 1.1137x over previous
import functools

import jax
import jax.numpy as jnp
from jax import lax
from jax.experimental import pallas as pl
from jax.experimental.pallas import tpu as pltpu, tpu_sc as plsc

B = 1024
V = 100000
H = V // 2
NC = 2
NS = 16
L = 16
NW = NC * NS
RPW = B // NW

_mesh = plsc.VectorSubcoreMesh(
    core_axis_name="c", subcore_axis_name="s", num_cores=NC, num_subcores=NS
)

_DN = lax.GatherDimensionNumbers(
    offset_dims=(), collapsed_slice_dims=(0,), start_index_map=(0,)
)


def _shuf(v, idx):
    return lax.gather(
        v, idx.reshape(L, 1), _DN, slice_sizes=(1,),
        mode=lax.GatherScatterMode.PROMISE_IN_BOUNDS,
    )


@functools.partial(
    pl.kernel,
    out_type=jax.ShapeDtypeStruct((B,), jnp.float32),
    mesh=_mesh,
    scratch_types=[
        pltpu.VMEM((H + L,), jnp.float32),
        pltpu.VMEM((H + L,), jnp.float32),
        pltpu.VMEM((RPW + L,), jnp.int32),
        pltpu.VMEM((RPW + L,), jnp.float32),
        pltpu.SemaphoreType.DMA,
        pltpu.SemaphoreType.DMA,
    ],
)
def _ll_kernel(in_hbm, tg_hbm, out_hbm, b0, b1, t_v, o_v, s0, s1):
    wid = lax.axis_index("s") * NC + lax.axis_index("c")
    base_row = wid * RPW
    lanes = lax.broadcasted_iota(jnp.int32, (L,), 0)
    zeros_i = lanes * 0
    neg_inf = jnp.full((L,), -jnp.inf, jnp.float32)

    pltpu.sync_copy(tg_hbm.at[pl.ds(base_row, RPW)], t_v.at[pl.ds(0, RPW)])

    def start(r, half, buf, sem):
        pltpu.make_async_copy(
            in_hbm.at[pl.ds((base_row + r) * V + half * H, H)],
            buf.at[pl.ds(0, H)],
            sem,
        ).start()

    def wait(buf, sem):
        pltpu.make_async_copy(
            in_hbm.at[pl.ds(0, H)], buf.at[pl.ds(0, H)], sem
        ).wait()

    start(0, 0, b0, s0)
    start(0, 1, b1, s1)

    def chunk_max(buf, acc):
        return lax.fori_loop(
            0,
            H // L,
            lambda i, a: jnp.maximum(a, buf[pl.ds(i * L, L)]),
            acc,
            unroll=8,
        )

    def fixup(buf, tl):
        s = tl.astype(jnp.uint32) < jnp.uint32(H)
        off = jnp.where(s, tl, 0)
        vec = buf[pl.ds(off, L)]
        v0 = vec[0]
        v0m = jnp.where(s, -jnp.inf, v0)
        buf[pl.ds(off, L)] = jnp.where(
            lanes == 0, jnp.broadcast_to(v0m, (L,)), vec
        )
        return jnp.broadcast_to(jnp.where(s, v0, 0.0), (L,))

    def do_row(r, carry):
        t = t_v[pl.ds(r, L)][0]
        wait(b0, s0)
        c0 = fixup(b0, t)
        acc = chunk_max(b0, neg_inf)

        @pl.when(r + 1 < RPW)
        def _():
            start(r + 1, 0, b0, s0)

        wait(b1, s1)
        c1 = fixup(b1, t - H)
        acc = chunk_max(b1, acc)

        @pl.when(r + 1 < RPW)
        def _():
            start(r + 1, 1, b1, s1)

        for k in (8, 4, 2, 1):
            acc = jnp.maximum(acc, _shuf(acc, lanes ^ k))
        o_v[pl.ds(r, L)] = (c0 + c1) - acc
        return carry

    lax.fori_loop(0, RPW, do_row, 0)
    pltpu.sync_copy(o_v.at[pl.ds(0, RPW)], out_hbm.at[pl.ds(base_row, RPW)])


def kernel(inputs, targets):
    return _ll_kernel(inputs.reshape(-1), targets.astype(jnp.int32))


# speedup vs baseline: 2.0966x; 1.8826x over previous
import functools

import jax
import jax.numpy as jnp
from jax import lax
from jax.experimental import pallas as pl
from jax.experimental.pallas import tpu as pltpu, tpu_sc as plsc

B = 1024
V = 100000
NC = 2
NS = 16
L = 16
NW = NC * NS
RPW = B // NW
TRW = RPW // 8
CW = 4096
VMAIN = 781 * 128
NCH = 25
LAST_COL = VMAIN - CW
JOBS = TRW * NCH
TAILC = V - VMAIN

_mesh = plsc.VectorSubcoreMesh(
    core_axis_name="c", subcore_axis_name="s", num_cores=NC, num_subcores=NS
)

_DN = lax.GatherDimensionNumbers(
    offset_dims=(), collapsed_slice_dims=(0,), start_index_map=(0,)
)


def _shuf(v, idx):
    return lax.gather(
        v, idx.reshape(L, 1), _DN, slice_sizes=(1,),
        mode=lax.GatherScatterMode.PROMISE_IN_BOUNDS,
    )


@functools.partial(
    pl.kernel,
    out_type=jax.ShapeDtypeStruct((B,), jnp.float32),
    mesh=_mesh,
    scratch_types=[
        pltpu.VMEM((8, CW), jnp.float32),
        pltpu.VMEM((8, CW), jnp.float32),
        pltpu.VMEM((RPW * 128,), jnp.float32),
        pltpu.VMEM((RPW + L,), jnp.int32),
        pltpu.VMEM((RPW + L,), jnp.float32),
        pltpu.VMEM((RPW + L,), jnp.float32),
        pltpu.VMEM((RPW + L,), jnp.float32),
        pltpu.SemaphoreType.DMA,
        pltpu.SemaphoreType.DMA,
    ],
)
def _ll_kernel(in_hbm, tg_hbm, tail_hbm, out_hbm,
               bufa, bufb, tl_v, t_v, o_v, ts_v, tc_v, sa, sb):
    wid = lax.axis_index("s") * NC + lax.axis_index("c")
    base_row = wid * RPW
    lanes = lax.broadcasted_iota(jnp.int32, (L,), 0)
    zeros_i = lanes * 0
    neg_inf = jnp.full((L,), -jnp.inf, jnp.float32)

    def job_src(j):
        b = j // NCH
        k = j - b * NCH
        is24 = jnp.where(k == NCH - 1, 1, 0)
        col = pl.multiple_of(CW * k - is24 * (CW * (NCH - 1) - LAST_COL), 128)
        return b, k, col

    def start(j, buf, sem):
        b, _, col = job_src(j)
        pltpu.make_async_copy(
            in_hbm.at[pl.ds(pl.multiple_of(base_row + b * 8, 8), 8),
                      pl.ds(col, CW)],
            buf, sem,
        ).start()

    def wait(buf, sem):
        pltpu.make_async_copy(
            in_hbm.at[pl.ds(0, 8), pl.ds(0, CW)], buf, sem
        ).wait()

    start(0, bufa, sa)
    start(1, bufb, sb)
    pltpu.sync_copy(tg_hbm.at[pl.ds(base_row, RPW)], t_v.at[pl.ds(0, RPW)])
    pltpu.sync_copy(tail_hbm.at[pl.ds(base_row * 128, RPW * 128)], tl_v)

    def tail_row(r, carry):
        t = t_v[pl.ds(r, L)][0]
        cl = t - VMAIN
        inch = cl.astype(jnp.uint32) < jnp.uint32(TAILC)
        clm = jnp.where(inch, cl, 0)
        off = (clm // L) * L
        tm = clm & (L - 1)
        base = r * 128
        vec = tl_v[pl.ds(base + off, L)]
        onehot = jnp.where(lanes == 0, 1.0, 0.0)
        inf = jnp.where(inch, 1.0, 0.0)
        oh = _shuf(onehot, (lanes - tm) & (L - 1)) * inf
        cval = _shuf(vec, jnp.broadcast_to(tm, (L,))) * inf
        tl_v[pl.ds(base + off, L)] = jnp.where(oh > 0.5, neg_inf, vec)
        acc = neg_inf
        for u in range(8):
            acc = jnp.maximum(acc, tl_v[pl.ds(base + u * L, L)])
        for k in (8, 4, 2, 1):
            acc = jnp.maximum(acc, _shuf(acc, lanes ^ k))
        ts_v[pl.ds(r, L)] = acc
        tc_v[pl.ds(r, L)] = cval
        return carry

    lax.fori_loop(0, RPW, tail_row, 0)

    def do_job(j, buf, sem, accs, cps):
        b, k, col = job_src(j)
        k0f = jnp.where(k == 0, 1.0, 0.0)
        nk0f = 1.0 - k0f
        is24 = jnp.where(k == NCH - 1, 1, 0)
        fix_lo = CW * k
        fix_w = CW - is24 * (CW * NCH - VMAIN)
        wait(buf, sem)
        naccs, ncps = [], []
        for s in range(8):
            rb = b * 8 + s
            t = t_v[pl.ds(rb, L)][0]
            seed = _shuf(ts_v[pl.ds(rb, L)], zeros_i)
            cseed = _shuf(tc_v[pl.ds(rb, L)], zeros_i)
            acc = accs[s] * nk0f + seed * k0f
            cp = cps[s] * nk0f + cseed * k0f
            cl = t - fix_lo
            inch = cl.astype(jnp.uint32) < fix_w.astype(jnp.uint32)
            inf = jnp.where(inch, 1.0, 0.0)
            blcm = jnp.where(inch, t - col, 0)
            off = (blcm // L) * L
            vec = buf[s, pl.ds(off, L)]
            tm = blcm & (L - 1)
            onehot = jnp.where(lanes == 0, 1.0, 0.0)
            oh = _shuf(onehot, (lanes - tm) & (L - 1)) * inf
            cp = cp + _shuf(vec, jnp.broadcast_to(tm, (L,))) * inf
            buf[s, pl.ds(off, L)] = jnp.where(oh > 0.5, neg_inf, vec)
            naccs.append(acc)
            ncps.append(cp)

        def hot(i, a):
            return tuple(
                jnp.maximum(a[s], buf[s, pl.ds(i * L, L)]) for s in range(8)
            )

        naccs = list(lax.fori_loop(0, CW // L, hot, tuple(naccs), unroll=2))

        @pl.when(j + 2 < JOBS)
        def _():
            start(j + 2, buf, sem)

        @pl.when(k == NCH - 1)
        def _():
            for s in range(8):
                acc = naccs[s]
                for kk in (8, 4, 2, 1):
                    acc = jnp.maximum(acc, _shuf(acc, lanes ^ kk))
                o_v[pl.ds(b * 8 + s, L)] = ncps[s] - acc

        return naccs, ncps

    def pair(m, carry):
        accs = list(carry[:8])
        cps = list(carry[8:])
        accs, cps = do_job(2 * m, bufa, sa, accs, cps)
        accs, cps = do_job(2 * m + 1, bufb, sb, accs, cps)
        return tuple(accs) + tuple(cps)

    init = tuple(jnp.zeros((L,), jnp.float32) for _ in range(16))
    lax.fori_loop(0, JOBS // 2, pair, init)
    pltpu.sync_copy(o_v.at[pl.ds(0, RPW)], out_hbm.at[pl.ds(base_row, RPW)])


def kernel(inputs, targets):
    tail = jnp.pad(
        inputs[:, VMAIN:], ((0, 0), (0, 128 - TAILC)),
        constant_values=-jnp.inf,
    ).reshape(-1)
    return _ll_kernel(inputs, targets.astype(jnp.int32), tail)


# speedup vs baseline: 5.7784x; 2.7562x over previous
import functools

import jax
import jax.numpy as jnp
from jax import lax
from jax.experimental import pallas as pl
from jax.experimental.pallas import tpu as pltpu, tpu_sc as plsc

B = 1024
V = 100000
NC = 2
NS = 16
L = 16
NW = NC * NS
NB = 8
NG = NW // NB
SW = V // NG
VCH = 200
NCHK = SW // VCH
BL = B // NB
NEG = float("-inf")

_mesh = plsc.VectorSubcoreMesh(
    core_axis_name="c", subcore_axis_name="s", num_cores=NC, num_subcores=NS
)


@functools.partial(
    pl.kernel,
    out_type=(
        jax.ShapeDtypeStruct((NG * B,), jnp.float32),
        jax.ShapeDtypeStruct((NG * B,), jnp.float32),
    ),
    mesh=_mesh,
    scratch_types=[
        pltpu.VMEM((VCH, BL), jnp.float32),
        pltpu.VMEM((VCH, BL), jnp.float32),
        pltpu.VMEM((BL,), jnp.float32),
        pltpu.VMEM((BL,), jnp.float32),
        pltpu.SemaphoreType.DMA,
        pltpu.SemaphoreType.DMA,
    ],
)
def _topk_kernel(xt_hbm, m1_hbm, m2_hbm, bufa, bufb, m1_v, m2_v, sa, sb):
    wid = lax.axis_index("s") * NC + lax.axis_index("c")
    jb = wid % NB
    gs = wid // NB
    v_lo = gs * SW
    b_lo = jb * BL

    def start(k, buf, sem):
        pltpu.make_async_copy(
            xt_hbm.at[
                pl.ds(pl.multiple_of(v_lo + k * VCH, 8), VCH),
                pl.ds(pl.multiple_of(b_lo, 128), BL),
            ],
            buf, sem,
        ).start()

    def wait(buf, sem):
        pltpu.make_async_copy(
            xt_hbm.at[pl.ds(0, VCH), pl.ds(0, BL)], buf, sem
        ).wait()

    start(0, bufa, sa)
    start(1, bufb, sb)

    def do_chunk(k, buf, sem, m1s, m2s):
        wait(buf, sem)

        def hot(i, carry):
            a = list(carry[:8])
            b = list(carry[8:])
            for u in range(8):
                x = buf[i, pl.ds(u * L, L)]
                b[u] = jnp.maximum(b[u], jnp.minimum(a[u], x))
                a[u] = jnp.maximum(a[u], x)
            return tuple(a) + tuple(b)

        out = lax.fori_loop(0, VCH, hot, tuple(m1s) + tuple(m2s), unroll=2)

        @pl.when(k + 2 < NCHK)
        def _():
            start(k + 2, buf, sem)

        return list(out[:8]), list(out[8:])

    neg = jnp.full((L,), NEG, jnp.float32)
    init = tuple(neg for _ in range(16))

    def pair(m, carry):
        m1s, m2s = list(carry[:8]), list(carry[8:])
        m1s, m2s = do_chunk(2 * m, bufa, sa, m1s, m2s)
        m1s, m2s = do_chunk(2 * m + 1, bufb, sb, m1s, m2s)
        return tuple(m1s) + tuple(m2s)

    fin = lax.fori_loop(0, NCHK // 2, pair, init)
    m1s, m2s = list(fin[:8]), list(fin[8:])
    m1s, m2s = do_chunk(NCHK - 1, bufa, sa, m1s, m2s)

    for u in range(8):
        m1_v[pl.ds(u * L, L)] = m1s[u]
        m2_v[pl.ds(u * L, L)] = m2s[u]
    pltpu.sync_copy(m1_v, m1_hbm.at[pl.ds(gs * B + b_lo, BL)])
    pltpu.sync_copy(m2_v, m2_hbm.at[pl.ds(gs * B + b_lo, BL)])


RPW = B // NW


@functools.partial(
    pl.kernel,
    out_type=jax.ShapeDtypeStruct((B,), jnp.float32),
    mesh=_mesh,
    scratch_types=[
        pltpu.VMEM((NG * RPW,), jnp.float32),
        pltpu.VMEM((NG * RPW,), jnp.float32),
        pltpu.VMEM((RPW,), jnp.float32),
        pltpu.VMEM((RPW,), jnp.float32),
    ],
)
def _combine_kernel(m1_hbm, m2_hbm, c_hbm, out_hbm, m1_v, m2_v, c_v, o_v):
    wid = lax.axis_index("s") * NC + lax.axis_index("c")
    b_lo = wid * RPW
    for gs in range(NG):
        pltpu.sync_copy(m1_hbm.at[pl.ds(gs * B + b_lo, RPW)],
                        m1_v.at[pl.ds(gs * RPW, RPW)])
        pltpu.sync_copy(m2_hbm.at[pl.ds(gs * B + b_lo, RPW)],
                        m2_v.at[pl.ds(gs * RPW, RPW)])
    pltpu.sync_copy(c_hbm.at[pl.ds(b_lo, RPW)], c_v)
    for g in range(RPW // L):
        M1 = m1_v[pl.ds(g * L, L)]
        M2 = m2_v[pl.ds(g * L, L)]
        for gs in range(1, NG):
            B1 = m1_v[pl.ds(gs * RPW + g * L, L)]
            B2 = m2_v[pl.ds(gs * RPW + g * L, L)]
            M2 = jnp.maximum(jnp.minimum(M1, B1), jnp.maximum(M2, B2))
            M1 = jnp.maximum(M1, B1)
        c = c_v[pl.ds(g * L, L)]
        o_v[pl.ds(g * L, L)] = c - jnp.where(c == M1, M2, M1)
    pltpu.sync_copy(o_v, out_hbm.at[pl.ds(b_lo, RPW)])


def kernel(inputs, targets):
    tg = targets.astype(jnp.int32)
    xt = inputs.T
    cls = jnp.take_along_axis(inputs, tg[:, None], axis=1)[:, 0]
    m1, m2 = _topk_kernel(xt)
    return _combine_kernel(m1, m2, cls)
